# baseline (speedup 1.0000x reference)
import functools

import jax
import jax.numpy as jnp
from jax import lax
from jax.experimental import pallas as pl
from jax.experimental.pallas import tpu as pltpu
from jax.experimental.pallas import tpu_sc as plsc

_B, _S, _H, _D, _E = 1, 2048, 1024, 3584, 8
_T = _B * _S
_NP = 2 * _T
_TILE = 256
_NT = _NP // _TILE + _E
_NSLOT = _NT * _TILE
_DC1 = 896
_ND1 = _D // _DC1
_DC2 = 896
_ND2 = _D // _DC2
_NW = 32
_PPW = _NP // _NW
_CH = 32
_NJ = _PPW // _CH
_TPW = _T // _NW
_CCH = 16
_NCJ = _TPW // _CCH


def _plan_body(x_ref, wg_ref, pos_ref, g16_ref, te_ref, m_ref, rank_ref):
    lg = lax.dot_general(x_ref[...], wg_ref[...], (((1,), (1,)), ((), ())),
                         preferred_element_type=jnp.float32)
    iota = lax.broadcasted_iota(jnp.int32, lg.shape, 1)
    i1 = jnp.argmax(lg, axis=1)[:, None]
    oh1 = iota == i1
    m1 = jnp.max(lg, axis=1, keepdims=True)
    lg2 = jnp.where(oh1, -jnp.inf, lg)
    i2 = jnp.argmax(lg2, axis=1)[:, None]
    oh2 = iota == i2
    m2 = jnp.max(lg2, axis=1, keepdims=True)
    p2 = jnp.exp(m2 - m1)
    denom = 1.0 + p2
    w1n = 1.0 / denom
    w2n = p2 / denom
    m_ref[...] = jnp.concatenate(
        [oh1.astype(jnp.float32), oh2.astype(jnp.float32)], axis=0)
    ones16 = jnp.ones((1, 128), jnp.float32)
    g16_ref[...] = jnp.concatenate([w1n * ones16, w2n * ones16], axis=0)

    r_iota = lax.broadcasted_iota(jnp.int32, (128, 128), 0)
    c_iota = lax.broadcasted_iota(jnp.int32, (128, 128), 1)
    tstrict = (r_iota > c_iota).astype(jnp.float32)

    def chunk(c, carry):
        mc = m_ref[pl.ds(c * 128, 128), :]
        prefix = lax.dot_general(tstrict, mc, (((1,), (0,)), ((), ())),
                                 preferred_element_type=jnp.float32)
        rank_ref[pl.ds(c * 128, 128), :] = prefix + carry
        return carry + jnp.sum(mc, axis=0, keepdims=True)

    counts = lax.fori_loop(0, _NP // 128, chunk,
                           jnp.zeros((1, _E), jnp.float32))
    pc = jnp.floor((counts + float(_TILE - 1)) / float(_TILE)) * float(_TILE)
    er = lax.broadcasted_iota(jnp.int32, (_E, _E), 0)
    ec = lax.broadcasted_iota(jnp.int32, (_E, _E), 1)
    ustrict = (er < ec).astype(jnp.float32)
    off = lax.dot_general(pc, ustrict, (((1,), (0,)), ((), ())),
                          preferred_element_type=jnp.float32)
    cum_end = (off + pc).astype(jnp.int32)
    posm = rank_ref[...] + off
    pos = jnp.sum(posm * m_ref[...], axis=1, keepdims=True)
    pos_ref[...] = pos.astype(jnp.int32)
    sb = lax.broadcasted_iota(jnp.int32, (_NT, _E), 0) * _TILE
    te = jnp.sum((sb >= cum_end).astype(jnp.int32), axis=1)
    te_ref[...] = jnp.minimum(te, _E - 1).reshape(1, _NT)


def _plan(xf, Wg):
    return pl.pallas_call(
        _plan_body,
        grid=(1,),
        in_specs=[pl.BlockSpec((_T, _H), lambda i: (0, 0)),
                  pl.BlockSpec((_E, _H), lambda i: (0, 0))],
        out_specs=[pl.BlockSpec((_NP, 1), lambda i: (0, 0)),
                   pl.BlockSpec((_NP, 128), lambda i: (0, 0)),
                   pl.BlockSpec((1, _NT), lambda i: (0, 0))],
        out_shape=[jax.ShapeDtypeStruct((_NP, 1), jnp.int32),
                   jax.ShapeDtypeStruct((_NP, 128), jnp.float32),
                   jax.ShapeDtypeStruct((1, _NT), jnp.int32)],
        scratch_shapes=[pltpu.VMEM((_NP, _E), jnp.float32),
                        pltpu.VMEM((_NP, _E), jnp.float32)],
    )(xf, Wg)


def _sc_dispatch(xb, pos3, g16):
    mesh = plsc.VectorSubcoreMesh(core_axis_name="c", subcore_axis_name="s")

    @functools.partial(
        pl.kernel, mesh=mesh,
        out_type=[jax.ShapeDtypeStruct((_NSLOT, _H // 2), jnp.int32),
                  jax.ShapeDtypeStruct((_NSLOT, 128), jnp.float32)],
        scratch_types=[pltpu.VMEM((_NJ, _CH), jnp.int32),
                       pltpu.VMEM((_CH, _H // 2), jnp.int32),
                       pltpu.VMEM((_CH, 128), jnp.float32),
                       pltpu.SemaphoreType.DMA,
                       pltpu.SemaphoreType.DMA],
    )
    def k(x_hbm, pos_hbm, g_hbm, xs_hbm, gs_hbm, posv, xbuf, gbuf, sem, sem2):
        wid = lax.axis_index("s") * 2 + lax.axis_index("c")
        pltpu.sync_copy(pos_hbm.at[wid], posv)
        for j in range(_NJ):
            base = wid * _PPW + j * _CH
            toff = lax.rem(base, _T)
            pltpu.sync_copy(x_hbm.at[pl.ds(toff, _CH)], xbuf)
            cp = pltpu.async_copy(xbuf, xs_hbm.at[posv.at[j]], sem)
            pltpu.sync_copy(g_hbm.at[pl.ds(base, _CH)], gbuf)
            cp2 = pltpu.async_copy(gbuf, gs_hbm.at[posv.at[j]], sem2)
            cp.wait()
            cp2.wait()

    return k(xb, pos3, g16)


def _c1_body(te_ref, xs_ref, w1_ref, w3_ref, gs_ref, h_ref):
    g = pl.program_id(1)
    xt = xs_ref[pl.ds(g * _TILE, _TILE), :].astype(jnp.float32)
    a1 = lax.dot_general(xt, w1_ref[0], (((1,), (1,)), ((), ())),
                         preferred_element_type=jnp.float32)
    a3 = lax.dot_general(xt, w3_ref[0], (((1,), (1,)), ((), ())),
                         preferred_element_type=jnp.float32)
    h = a1 * jax.nn.sigmoid(a1) * a3 * gs_ref[:, 0:1]
    h_ref[...] = h.astype(jnp.bfloat16)


def _c1(tef, xs, W1, W3, gs16):
    grid_spec = pltpu.PrefetchScalarGridSpec(
        num_scalar_prefetch=1,
        grid=(_ND1, _NT),
        in_specs=[
            pl.BlockSpec((_NSLOT, _H), lambda d, g, te: (0, 0)),
            pl.BlockSpec((1, _DC1, _H), lambda d, g, te: (te[g], d, 0)),
            pl.BlockSpec((1, _DC1, _H), lambda d, g, te: (te[g], d, 0)),
            pl.BlockSpec((_TILE, 128), lambda d, g, te: (g, 0)),
        ],
        out_specs=pl.BlockSpec((_TILE, _DC1), lambda d, g, te: (g, d)),
    )
    return pl.pallas_call(
        _c1_body,
        grid_spec=grid_spec,
        out_shape=jax.ShapeDtypeStruct((_NSLOT, _D), jnp.bfloat16),
        compiler_params=pltpu.CompilerParams(
            dimension_semantics=("arbitrary", "arbitrary")),
    )(tef, xs, W1, W3, gs16)


def _c2_body(te_ref, h_ref, w2_ref, ys_ref):
    d2 = pl.program_id(1)
    part = lax.dot_general(h_ref[...].astype(jnp.float32), w2_ref[0],
                           (((1,), (1,)), ((), ())),
                           preferred_element_type=jnp.float32)

    @pl.when(d2 == 0)
    def _set():
        ys_ref[...] = part

    @pl.when(d2 != 0)
    def _acc():
        ys_ref[...] += part


def _c2(tef, h, W2):
    grid_spec = pltpu.PrefetchScalarGridSpec(
        num_scalar_prefetch=1,
        grid=(_NT, _ND2),
        in_specs=[
            pl.BlockSpec((_TILE, _DC2), lambda g, d2, te: (g, d2)),
            pl.BlockSpec((1, _H, _DC2), lambda g, d2, te: (te[g], 0, d2)),
        ],
        out_specs=pl.BlockSpec((_TILE, _H), lambda g, d2, te: (g, 0)),
    )
    return pl.pallas_call(
        _c2_body,
        grid_spec=grid_spec,
        out_shape=jax.ShapeDtypeStruct((_NSLOT, _H), jnp.float32),
        compiler_params=pltpu.CompilerParams(
            dimension_semantics=("arbitrary", "arbitrary")),
    )(tef, h, W2)


def _sc_combine(ys, posf):
    mesh = plsc.VectorSubcoreMesh(core_axis_name="c", subcore_axis_name="s")

    @functools.partial(
        pl.kernel, mesh=mesh,
        out_type=jax.ShapeDtypeStruct((_T, _H), jnp.float32),
        scratch_types=[pltpu.VMEM((_CCH,), jnp.int32),
                       pltpu.VMEM((_CCH,), jnp.int32),
                       pltpu.VMEM((_CCH, _H), jnp.float32),
                       pltpu.VMEM((_CCH, _H), jnp.float32),
                       pltpu.SemaphoreType.DMA,
                       pltpu.SemaphoreType.DMA],
    )
    def k(ys_hbm, pos_hbm, out_hbm, idx0, idx1, buf0, buf1, sem, sem2):
        wid = lax.axis_index("s") * 2 + lax.axis_index("c")
        for j in range(_NCJ):
            base = wid * _TPW + j * _CCH
            pltpu.sync_copy(pos_hbm.at[pl.ds(base, _CCH)], idx0)
            pltpu.sync_copy(pos_hbm.at[pl.ds(_T + base, _CCH)], idx1)
            cp0 = pltpu.async_copy(ys_hbm.at[idx0], buf0, sem)
            cp1 = pltpu.async_copy(ys_hbm.at[idx1], buf1, sem2)
            cp0.wait()
            cp1.wait()

            def cbody(ci, _):
                col = ci * 16
                for r in range(_CCH):
                    buf0[r, pl.ds(col, 16)] = (buf0[r, pl.ds(col, 16)]
                                               + buf1[r, pl.ds(col, 16)])
                return 0

            lax.fori_loop(0, _H // 16, cbody, 0)
            pltpu.sync_copy(buf0, out_hbm.at[pl.ds(base, _CCH)])

    return k(ys, posf)


def kernel(x, Wg, W1, W2, W3):
    xf = x.reshape(_T, _H)
    xb = xf.astype(jnp.bfloat16)
    pos, g16, te = _plan(xf, Wg)
    pos3 = pos.reshape(_NW, _NJ, _CH)
    posf = pos.reshape(_NP)
    tef = te.reshape(_NT)
    xq = lax.bitcast_convert_type(xb.reshape(_T, _H // 2, 2), jnp.int32)
    xsq, gs16 = _sc_dispatch(xq, pos3, g16)
    xs = lax.bitcast_convert_type(xsq, jnp.bfloat16).reshape(_NSLOT, _H)
    h = _c1(tef, xs, W1, W3, gs16)
    ys = _c2(tef, h, W2)
    out = _sc_combine(ys, posf)
    return out.reshape(_B, _S, _H)


# speedup vs baseline: 1.3853x; 1.3853x over previous
import functools

import jax
import jax.numpy as jnp
from jax import lax
from jax.experimental import pallas as pl
from jax.experimental.pallas import tpu as pltpu
from jax.experimental.pallas import tpu_sc as plsc

_B, _S, _H, _D, _E = 1, 2048, 1024, 3584, 8
_T = _B * _S
_NP = 2 * _T
_TILE = 256
_NT = _NP // _TILE + _E
_NSLOT = _NT * _TILE
_DC1 = 896
_ND1 = _D // _DC1
_DC2 = 896
_ND2 = _D // _DC2
_NW = 32
_PPW = _NP // _NW
_CH = 32
_NJ = _PPW // _CH
_TPW = _T // _NW
_CCH = 16
_NCJ = _TPW // _CCH


def _plan_body(x_ref, wg_ref, pos_ref, g16_ref, te_ref, m_ref, rank_ref):
    lg = lax.dot_general(x_ref[...], wg_ref[...], (((1,), (1,)), ((), ())),
                         preferred_element_type=jnp.float32)
    iota = lax.broadcasted_iota(jnp.int32, lg.shape, 1)
    i1 = jnp.argmax(lg, axis=1)[:, None]
    oh1 = iota == i1
    m1 = jnp.max(lg, axis=1, keepdims=True)
    lg2 = jnp.where(oh1, -jnp.inf, lg)
    i2 = jnp.argmax(lg2, axis=1)[:, None]
    oh2 = iota == i2
    m2 = jnp.max(lg2, axis=1, keepdims=True)
    p2 = jnp.exp(m2 - m1)
    denom = 1.0 + p2
    w1n = 1.0 / denom
    w2n = p2 / denom
    m_ref[...] = jnp.concatenate(
        [oh1.astype(jnp.float32), oh2.astype(jnp.float32)], axis=0)
    ones16 = jnp.ones((1, 128), jnp.float32)
    g16_ref[...] = jnp.concatenate([w1n * ones16, w2n * ones16], axis=0)

    r_iota = lax.broadcasted_iota(jnp.int32, (128, 128), 0)
    c_iota = lax.broadcasted_iota(jnp.int32, (128, 128), 1)
    tstrict = (r_iota > c_iota).astype(jnp.float32)

    def chunk(c, carry):
        mc = m_ref[pl.ds(c * 128, 128), :]
        prefix = lax.dot_general(tstrict, mc, (((1,), (0,)), ((), ())),
                                 preferred_element_type=jnp.float32)
        rank_ref[pl.ds(c * 128, 128), :] = prefix + carry
        return carry + jnp.sum(mc, axis=0, keepdims=True)

    counts = lax.fori_loop(0, _NP // 128, chunk,
                           jnp.zeros((1, _E), jnp.float32))
    pc = jnp.floor((counts + float(_TILE - 1)) / float(_TILE)) * float(_TILE)
    er = lax.broadcasted_iota(jnp.int32, (_E, _E), 0)
    ec = lax.broadcasted_iota(jnp.int32, (_E, _E), 1)
    ustrict = (er < ec).astype(jnp.float32)
    off = lax.dot_general(pc, ustrict, (((1,), (0,)), ((), ())),
                          preferred_element_type=jnp.float32)
    cum_end = (off + pc).astype(jnp.int32)
    posm = rank_ref[...] + off
    pos = jnp.sum(posm * m_ref[...], axis=1, keepdims=True)
    pos_ref[...] = pos.astype(jnp.int32)
    sb = lax.broadcasted_iota(jnp.int32, (_NT, _E), 0) * _TILE
    te = jnp.sum((sb >= cum_end).astype(jnp.int32), axis=1)
    te_ref[...] = jnp.minimum(te, _E - 1).reshape(1, _NT)


def _plan(xf, Wg):
    return pl.pallas_call(
        _plan_body,
        grid=(1,),
        in_specs=[pl.BlockSpec((_T, _H), lambda i: (0, 0)),
                  pl.BlockSpec((_E, _H), lambda i: (0, 0))],
        out_specs=[pl.BlockSpec((_NP, 1), lambda i: (0, 0)),
                   pl.BlockSpec((_NP, 128), lambda i: (0, 0)),
                   pl.BlockSpec((1, _NT), lambda i: (0, 0))],
        out_shape=[jax.ShapeDtypeStruct((_NP, 1), jnp.int32),
                   jax.ShapeDtypeStruct((_NP, 128), jnp.float32),
                   jax.ShapeDtypeStruct((1, _NT), jnp.int32)],
        scratch_shapes=[pltpu.VMEM((_NP, _E), jnp.float32),
                        pltpu.VMEM((_NP, _E), jnp.float32)],
    )(xf, Wg)


def _sc_dispatch(xb, pos3, g16):
    mesh = plsc.VectorSubcoreMesh(core_axis_name="c", subcore_axis_name="s")

    @functools.partial(
        pl.kernel, mesh=mesh,
        out_type=[jax.ShapeDtypeStruct((_NSLOT, _H), jnp.float32),
                  jax.ShapeDtypeStruct((_NSLOT, 128), jnp.float32)],
        scratch_types=[pltpu.VMEM((_NJ, _CH), jnp.int32),
                       pltpu.VMEM((_CH, _H), jnp.float32),
                       pltpu.VMEM((_CH, 128), jnp.float32),
                       pltpu.SemaphoreType.DMA,
                       pltpu.SemaphoreType.DMA],
    )
    def k(x_hbm, pos_hbm, g_hbm, xs_hbm, gs_hbm, posv, xbuf, gbuf, sem, sem2):
        wid = lax.axis_index("s") * 2 + lax.axis_index("c")
        pltpu.sync_copy(pos_hbm.at[wid], posv)
        for j in range(_NJ):
            base = wid * _PPW + j * _CH
            toff = lax.rem(base, _T)
            ld0 = pltpu.async_copy(x_hbm.at[pl.ds(toff, _CH)], xbuf, sem)
            ld1 = pltpu.async_copy(g_hbm.at[pl.ds(base, _CH)], gbuf, sem2)
            ld0.wait()
            ld1.wait()
            cp = pltpu.async_copy(xbuf, xs_hbm.at[posv.at[j]], sem)
            cp2 = pltpu.async_copy(gbuf, gs_hbm.at[posv.at[j]], sem2)
            cp.wait()
            cp2.wait()

    return k(xb, pos3, g16)


def _c1_body(te_ref, xs_ref, w1_ref, w3_ref, gs_ref, h_ref):
    xt = xs_ref[...]
    a1 = lax.dot_general(xt, w1_ref[0], (((1,), (1,)), ((), ())),
                         preferred_element_type=jnp.float32)
    a3 = lax.dot_general(xt, w3_ref[0], (((1,), (1,)), ((), ())),
                         preferred_element_type=jnp.float32)
    h = a1 * jax.nn.sigmoid(a1) * a3 * gs_ref[:, 0:1]
    h_ref[...] = h.astype(jnp.bfloat16)


def _c1(tef, xs, W1, W3, gs16):
    grid_spec = pltpu.PrefetchScalarGridSpec(
        num_scalar_prefetch=1,
        grid=(_ND1, _NT),
        in_specs=[
            pl.BlockSpec((_TILE, _H), lambda d, g, te: (g, 0)),
            pl.BlockSpec((1, _DC1, _H), lambda d, g, te: (te[g], d, 0)),
            pl.BlockSpec((1, _DC1, _H), lambda d, g, te: (te[g], d, 0)),
            pl.BlockSpec((_TILE, 128), lambda d, g, te: (g, 0)),
        ],
        out_specs=pl.BlockSpec((_TILE, _DC1), lambda d, g, te: (g, d)),
    )
    return pl.pallas_call(
        _c1_body,
        grid_spec=grid_spec,
        out_shape=jax.ShapeDtypeStruct((_NSLOT, _D), jnp.bfloat16),
        compiler_params=pltpu.CompilerParams(
            dimension_semantics=("arbitrary", "arbitrary")),
    )(tef, xs, W1, W3, gs16)


def _c2_body(te_ref, h_ref, w2_ref, ys_ref):
    d2 = pl.program_id(1)
    part = lax.dot_general(h_ref[...].astype(jnp.float32), w2_ref[0],
                           (((1,), (1,)), ((), ())),
                           preferred_element_type=jnp.float32)

    @pl.when(d2 == 0)
    def _set():
        ys_ref[...] = part

    @pl.when(d2 != 0)
    def _acc():
        ys_ref[...] += part


def _c2(tef, h, W2):
    grid_spec = pltpu.PrefetchScalarGridSpec(
        num_scalar_prefetch=1,
        grid=(_NT, _ND2),
        in_specs=[
            pl.BlockSpec((_TILE, _DC2), lambda g, d2, te: (g, d2)),
            pl.BlockSpec((1, _H, _DC2), lambda g, d2, te: (te[g], 0, d2)),
        ],
        out_specs=pl.BlockSpec((_TILE, _H), lambda g, d2, te: (g, 0)),
    )
    return pl.pallas_call(
        _c2_body,
        grid_spec=grid_spec,
        out_shape=jax.ShapeDtypeStruct((_NSLOT, _H), jnp.float32),
        compiler_params=pltpu.CompilerParams(
            dimension_semantics=("arbitrary", "arbitrary")),
    )(tef, h, W2)


def _sc_combine(ys, posf):
    mesh = plsc.VectorSubcoreMesh(core_axis_name="c", subcore_axis_name="s")

    @functools.partial(
        pl.kernel, mesh=mesh,
        out_type=jax.ShapeDtypeStruct((_T, _H), jnp.float32),
        scratch_types=[pltpu.VMEM((_CCH,), jnp.int32),
                       pltpu.VMEM((_CCH,), jnp.int32),
                       pltpu.VMEM((_CCH, _H), jnp.float32),
                       pltpu.VMEM((_CCH, _H), jnp.float32),
                       pltpu.SemaphoreType.DMA,
                       pltpu.SemaphoreType.DMA],
    )
    def k(ys_hbm, pos_hbm, out_hbm, idx0, idx1, buf0, buf1, sem, sem2):
        wid = lax.axis_index("s") * 2 + lax.axis_index("c")
        for j in range(_NCJ):
            base = wid * _TPW + j * _CCH
            pltpu.sync_copy(pos_hbm.at[pl.ds(base, _CCH)], idx0)
            pltpu.sync_copy(pos_hbm.at[pl.ds(_T + base, _CCH)], idx1)
            cp0 = pltpu.async_copy(ys_hbm.at[idx0], buf0, sem)
            cp1 = pltpu.async_copy(ys_hbm.at[idx1], buf1, sem2)
            cp0.wait()
            cp1.wait()

            def cbody(ci, _):
                col = ci * 16
                for r in range(_CCH):
                    buf0[r, pl.ds(col, 16)] = (buf0[r, pl.ds(col, 16)]
                                               + buf1[r, pl.ds(col, 16)])
                return 0

            lax.fori_loop(0, _H // 16, cbody, 0)
            pltpu.sync_copy(buf0, out_hbm.at[pl.ds(base, _CCH)])

    return k(ys, posf)


def kernel(x, Wg, W1, W2, W3):
    xf = x.reshape(_T, _H)
    pos, g16, te = _plan(xf, Wg)
    pos3 = pos.reshape(_NW, _NJ, _CH)
    posf = pos.reshape(_NP)
    tef = te.reshape(_NT)
    xs, gs16 = _sc_dispatch(xf, pos3, g16)
    h = _c1(tef, xs, W1, W3, gs16)
    ys = _c2(tef, h, W2)
    out = _sc_combine(ys, posf)
    return out.reshape(_B, _S, _H)


# speedup vs baseline: 1.5910x; 1.1485x over previous
import functools

import jax
import jax.numpy as jnp
from jax import lax
from jax.experimental import pallas as pl
from jax.experimental.pallas import tpu as pltpu
from jax.experimental.pallas import tpu_sc as plsc

_B, _S, _H, _D, _E = 1, 2048, 1024, 3584, 8
_T = _B * _S
_NP = 2 * _T
_TILE = 256
_NT = _NP // _TILE + _E
_NSLOT = _NT * _TILE
_DC1 = 896
_ND1 = _D // _DC1
_DC2 = 896
_ND2 = _D // _DC2
_NW = 32
_PPW = _NP // _NW
_CH = 32
_NJ = _PPW // _CH
_TPW = _T // _NW
_CCH = 16
_NCJ = _TPW // _CCH


def _plan_body(x_ref, wg_ref, pos_ref, g16_ref, te_ref, m_ref, rank_ref):
    lg = lax.dot_general(x_ref[...], wg_ref[...], (((1,), (1,)), ((), ())),
                         preferred_element_type=jnp.float32)
    iota = lax.broadcasted_iota(jnp.int32, lg.shape, 1)
    i1 = jnp.argmax(lg, axis=1)[:, None]
    oh1 = iota == i1
    m1 = jnp.max(lg, axis=1, keepdims=True)
    lg2 = jnp.where(oh1, -jnp.inf, lg)
    i2 = jnp.argmax(lg2, axis=1)[:, None]
    oh2 = iota == i2
    m2 = jnp.max(lg2, axis=1, keepdims=True)
    p2 = jnp.exp(m2 - m1)
    denom = 1.0 + p2
    w1n = 1.0 / denom
    w2n = p2 / denom
    m_ref[...] = jnp.concatenate(
        [oh1.astype(jnp.float32), oh2.astype(jnp.float32)], axis=0)
    ones16 = jnp.ones((1, 128), jnp.float32)
    g16_ref[...] = jnp.concatenate([w1n * ones16, w2n * ones16], axis=0)

    r_iota = lax.broadcasted_iota(jnp.int32, (128, 128), 0)
    c_iota = lax.broadcasted_iota(jnp.int32, (128, 128), 1)
    tstrict = (r_iota > c_iota).astype(jnp.float32)

    def chunk(c, carry):
        mc = m_ref[pl.ds(c * 128, 128), :]
        prefix = lax.dot_general(tstrict, mc, (((1,), (0,)), ((), ())),
                                 preferred_element_type=jnp.float32)
        rank_ref[pl.ds(c * 128, 128), :] = prefix + carry
        return carry + jnp.sum(mc, axis=0, keepdims=True)

    counts = lax.fori_loop(0, _NP // 128, chunk,
                           jnp.zeros((1, _E), jnp.float32))
    pc = jnp.floor((counts + float(_TILE - 1)) / float(_TILE)) * float(_TILE)
    er = lax.broadcasted_iota(jnp.int32, (_E, _E), 0)
    ec = lax.broadcasted_iota(jnp.int32, (_E, _E), 1)
    ustrict = (er < ec).astype(jnp.float32)
    off = lax.dot_general(pc, ustrict, (((1,), (0,)), ((), ())),
                          preferred_element_type=jnp.float32)
    cum_end = (off + pc).astype(jnp.int32)
    posm = rank_ref[...] + off
    pos = jnp.sum(posm * m_ref[...], axis=1, keepdims=True)
    pos_ref[...] = pos.astype(jnp.int32)
    sb = lax.broadcasted_iota(jnp.int32, (_NT, _E), 0) * _TILE
    te = jnp.sum((sb >= cum_end).astype(jnp.int32), axis=1)
    te_ref[...] = jnp.minimum(te, _E - 1).reshape(1, _NT)


def _plan(xf, Wg):
    return pl.pallas_call(
        _plan_body,
        grid=(1,),
        in_specs=[pl.BlockSpec((_T, _H), lambda i: (0, 0)),
                  pl.BlockSpec((_E, _H), lambda i: (0, 0))],
        out_specs=[pl.BlockSpec((_NP, 1), lambda i: (0, 0)),
                   pl.BlockSpec((_NP, 128), lambda i: (0, 0)),
                   pl.BlockSpec((1, _NT), lambda i: (0, 0))],
        out_shape=[jax.ShapeDtypeStruct((_NP, 1), jnp.int32),
                   jax.ShapeDtypeStruct((_NP, 128), jnp.float32),
                   jax.ShapeDtypeStruct((1, _NT), jnp.int32)],
        scratch_shapes=[pltpu.VMEM((_NP, _E), jnp.float32),
                        pltpu.VMEM((_NP, _E), jnp.float32)],
    )(xf, Wg)


def _sc_dispatch(xb, pos3, g16):
    mesh = plsc.VectorSubcoreMesh(core_axis_name="c", subcore_axis_name="s")

    @functools.partial(
        pl.kernel, mesh=mesh,
        out_type=[jax.ShapeDtypeStruct((_NSLOT, _H), jnp.float32),
                  jax.ShapeDtypeStruct((_NSLOT, 128), jnp.float32)],
        scratch_types=[pltpu.VMEM((_NJ, _CH), jnp.int32),
                       pltpu.VMEM((_CH, _H), jnp.float32),
                       pltpu.VMEM((_CH, 128), jnp.float32),
                       pltpu.SemaphoreType.DMA,
                       pltpu.SemaphoreType.DMA],
    )
    def k(x_hbm, pos_hbm, g_hbm, xs_hbm, gs_hbm, posv, xbuf, gbuf, sem, sem2):
        wid = lax.axis_index("s") * 2 + lax.axis_index("c")
        pltpu.sync_copy(pos_hbm.at[wid], posv)
        for j in range(_NJ):
            base = wid * _PPW + j * _CH
            toff = lax.rem(base, _T)
            ld0 = pltpu.async_copy(x_hbm.at[pl.ds(toff, _CH)], xbuf, sem)
            ld1 = pltpu.async_copy(g_hbm.at[pl.ds(base, _CH)], gbuf, sem2)
            ld0.wait()
            ld1.wait()
            cp = pltpu.async_copy(xbuf, xs_hbm.at[posv.at[j]], sem)
            cp2 = pltpu.async_copy(gbuf, gs_hbm.at[posv.at[j]], sem2)
            cp.wait()
            cp2.wait()

    return k(xb, pos3, g16)


def _c1_body(te_ref, xs_ref, w1_ref, w3_ref, h_ref):
    xt = xs_ref[...]
    a1 = lax.dot_general(xt, w1_ref[0], (((1,), (1,)), ((), ())),
                         preferred_element_type=jnp.float32)
    a3 = lax.dot_general(xt, w3_ref[0], (((1,), (1,)), ((), ())),
                         preferred_element_type=jnp.float32)
    h_ref[...] = a1 * jax.nn.sigmoid(a1) * a3


def _c1(tef, xs, W1, W3):
    grid_spec = pltpu.PrefetchScalarGridSpec(
        num_scalar_prefetch=1,
        grid=(_ND1, _NT),
        in_specs=[
            pl.BlockSpec((_TILE, _H), lambda d, g, te: (g, 0)),
            pl.BlockSpec((1, _DC1, _H), lambda d, g, te: (te[g], d, 0)),
            pl.BlockSpec((1, _DC1, _H), lambda d, g, te: (te[g], d, 0)),
        ],
        out_specs=pl.BlockSpec((_TILE, _DC1), lambda d, g, te: (g, d)),
    )
    return pl.pallas_call(
        _c1_body,
        grid_spec=grid_spec,
        out_shape=jax.ShapeDtypeStruct((_NSLOT, _D), jnp.float32),
        compiler_params=pltpu.CompilerParams(
            dimension_semantics=("arbitrary", "arbitrary")),
    )(tef, xs, W1, W3)


def _c2_body(te_ref, h_ref, w2_ref, gs_ref, ys_ref):
    y = lax.dot_general(h_ref[...], w2_ref[0], (((1,), (1,)), ((), ())),
                        preferred_element_type=jnp.float32)
    ys_ref[...] = y * gs_ref[:, 0:1]


def _c2(tef, h, W2, gs16):
    grid_spec = pltpu.PrefetchScalarGridSpec(
        num_scalar_prefetch=1,
        grid=(_NT,),
        in_specs=[
            pl.BlockSpec((_TILE, _D), lambda g, te: (g, 0)),
            pl.BlockSpec((1, _H, _D), lambda g, te: (te[g], 0, 0)),
            pl.BlockSpec((_TILE, 128), lambda g, te: (g, 0)),
        ],
        out_specs=pl.BlockSpec((_TILE, _H), lambda g, te: (g, 0)),
    )
    return pl.pallas_call(
        _c2_body,
        grid_spec=grid_spec,
        out_shape=jax.ShapeDtypeStruct((_NSLOT, _H), jnp.float32),
        compiler_params=pltpu.CompilerParams(
            dimension_semantics=("arbitrary",)),
    )(tef, h, W2, gs16)


def _sc_combine(ys, posf):
    mesh = plsc.VectorSubcoreMesh(core_axis_name="c", subcore_axis_name="s")

    @functools.partial(
        pl.kernel, mesh=mesh,
        out_type=jax.ShapeDtypeStruct((_T, _H), jnp.float32),
        scratch_types=[pltpu.VMEM((_CCH,), jnp.int32),
                       pltpu.VMEM((_CCH,), jnp.int32),
                       pltpu.VMEM((_CCH, _H), jnp.float32),
                       pltpu.VMEM((_CCH, _H), jnp.float32),
                       pltpu.SemaphoreType.DMA,
                       pltpu.SemaphoreType.DMA],
    )
    def k(ys_hbm, pos_hbm, out_hbm, idx0, idx1, buf0, buf1, sem, sem2):
        wid = lax.axis_index("s") * 2 + lax.axis_index("c")
        for j in range(_NCJ):
            base = wid * _TPW + j * _CCH
            pltpu.sync_copy(pos_hbm.at[pl.ds(base, _CCH)], idx0)
            pltpu.sync_copy(pos_hbm.at[pl.ds(_T + base, _CCH)], idx1)
            cp0 = pltpu.async_copy(ys_hbm.at[idx0], buf0, sem)
            cp1 = pltpu.async_copy(ys_hbm.at[idx1], buf1, sem2)
            cp0.wait()
            cp1.wait()

            def cbody(ci, _):
                col = ci * 16
                for r in range(_CCH):
                    buf0[r, pl.ds(col, 16)] = (buf0[r, pl.ds(col, 16)]
                                               + buf1[r, pl.ds(col, 16)])
                return 0

            lax.fori_loop(0, _H // 16, cbody, 0)
            pltpu.sync_copy(buf0, out_hbm.at[pl.ds(base, _CCH)])

    return k(ys, posf)


def kernel(x, Wg, W1, W2, W3):
    xf = x.reshape(_T, _H)
    pos, g16, te = _plan(xf, Wg)
    pos3 = pos.reshape(_NW, _NJ, _CH)
    posf = pos.reshape(_NP)
    tef = te.reshape(_NT)
    xs, gs16 = _sc_dispatch(xf, pos3, g16)
    h = _c1(tef, xs, W1, W3)
    ys = _c2(tef, h, W2, gs16)
    out = _sc_combine(ys, posf)
    return out.reshape(_B, _S, _H)
